# Initial kernel scaffold; baseline (speedup 1.0000x reference)
import jax
import jax.numpy as jnp
from jax.experimental import pallas as pl


def kernel(im_info, gt_boxes):
    raise NotImplementedError("write your pallas kernel here")


# speedup vs baseline: 5.7975x; 5.7975x over previous
import numpy as np
import jax
import jax.numpy as jnp
from jax.experimental import pallas as pl
from jax.experimental.pallas import tpu as pltpu

_FEATURES_SHAPE = (50, 50)
_STRIDE = 16
_ANCHOR_SIZE = 16
_NUM_GT = 64
_NEG_OVL = 0.4
_POS_OVL = 0.5


def _gen_anchors(base_size=16):
    ratios = np.array([0.5, 1.0, 2.0])
    scales = np.array([8.0, 16.0, 32.0])
    base = np.array([0.0, 0.0, base_size - 1.0, base_size - 1.0])
    w = base[2] - base[0] + 1.0
    h = base[3] - base[1] + 1.0
    x_ctr = base[0] + 0.5 * (w - 1.0)
    y_ctr = base[1] + 0.5 * (h - 1.0)
    size = w * h
    size_ratios = size / ratios
    ws = np.round(np.sqrt(size_ratios))
    hs = np.round(ws * ratios)

    def _mk(ws, hs, x_ctr, y_ctr):
        ws = ws[:, None]
        hs = hs[:, None]
        return np.hstack([x_ctr - 0.5 * (ws - 1.0), y_ctr - 0.5 * (hs - 1.0),
                          x_ctr + 0.5 * (ws - 1.0), y_ctr + 0.5 * (hs - 1.0)])

    ratio_anchors = _mk(ws, hs, x_ctr, y_ctr)
    out = []
    for i in range(ratio_anchors.shape[0]):
        a = ratio_anchors[i]
        w = a[2] - a[0] + 1.0
        h = a[3] - a[1] + 1.0
        x_ctr = a[0] + 0.5 * (w - 1.0)
        y_ctr = a[1] + 0.5 * (h - 1.0)
        out.append(_mk(w * scales, h * scales, x_ctr, y_ctr))
    return np.vstack(out).astype(np.float32)


def _shift_anchors(shape, stride, anchors):
    sx = np.arange(shape[1]) * stride
    sy = np.arange(shape[0]) * stride
    SX, SY = np.meshgrid(sx, sy)
    shifts = np.stack([SX.ravel(), SY.ravel(), SX.ravel(), SY.ravel()], axis=1)
    return (anchors[None, :, :] + shifts[:, None, :]).reshape(-1, 4).astype(np.float32)


_ANCHORS_NP = _shift_anchors(_FEATURES_SHAPE, _STRIDE, _gen_anchors(_ANCHOR_SIZE))
_N = _ANCHORS_NP.shape[0]
_ROWS, _LANES = 176, 128
_NPAD = _ROWS * _LANES

_PAD_BOX = np.array([0.0, 0.0, 15.0, 15.0], dtype=np.float32)
_A_PLANES = []
for _c in range(4):
    _v = np.full((_NPAD,), _PAD_BOX[_c], dtype=np.float32)
    _v[:_N] = _ANCHORS_NP[:, _c]
    _A_PLANES.append(_v.reshape(_ROWS, _LANES))
_AX1, _AY1, _AX2, _AY2 = (jnp.asarray(p) for p in _A_PLANES)


def _tc_body(gt_ref, im_ref, ax1_ref, ay1_ref, ax2_ref, ay2_ref,
             lab_ref, dx_ref, dy_ref, dw_ref, dh_ref):
    ax1 = ax1_ref[...]
    ay1 = ay1_ref[...]
    ax2 = ax2_ref[...]
    ay2 = ay2_ref[...]
    ex_w = ax2 - ax1 + 1.0
    ex_h = ay2 - ay1 + 1.0
    area_a = ex_w * ex_h

    def step(j, carry):
        best, bx1, by1, bx2, by2, bcls = carry
        gx1 = gt_ref[j, 0]
        gy1 = gt_ref[j, 1]
        gx2 = gt_ref[j, 2]
        gy2 = gt_ref[j, 3]
        gcls = gt_ref[j, 4]
        iw = jnp.maximum(jnp.minimum(ax2, gx2) - jnp.maximum(ax1, gx1) + 1.0, 0.0)
        ih = jnp.maximum(jnp.minimum(ay2, gy2) - jnp.maximum(ay1, gy1) + 1.0, 0.0)
        inter = iw * ih
        area_b = (gx2 - gx1 + 1.0) * (gy2 - gy1 + 1.0)
        union = area_a + area_b - inter
        iou = inter / union
        upd = iou > best
        best = jnp.where(upd, iou, best)
        bx1 = jnp.where(upd, gx1, bx1)
        by1 = jnp.where(upd, gy1, by1)
        bx2 = jnp.where(upd, gx2, bx2)
        by2 = jnp.where(upd, gy2, by2)
        bcls = jnp.where(upd, gcls, bcls)
        return best, bx1, by1, bx2, by2, bcls

    z = jnp.zeros((_ROWS, _LANES), jnp.float32)
    best, bx1, by1, bx2, by2, bcls = jax.lax.fori_loop(
        0, _NUM_GT, step, (z - 1.0, z, z, z, z, z))

    img_h = im_ref[0, 0]
    img_w = im_ref[0, 1]
    lab = z - 1.0
    lab = jnp.where(best < _NEG_OVL, z, lab)
    lab = jnp.where(best >= _POS_OVL, z + 1.0, lab)
    inside = (ax1 >= 0.0) & (ay1 >= 0.0) & (ax2 < img_w) & (ay2 < img_h)
    lab = jnp.where(inside, lab, z - 1.0)
    lab = jnp.where(lab == 1.0, bcls, lab)
    lab_ref[...] = lab

    gt_w = bx2 - bx1 + 1.0
    gt_h = by2 - by1 + 1.0
    ex_cx = ax1 + 0.5 * ex_w
    ex_cy = ay1 + 0.5 * ex_h
    gt_cx = bx1 + 0.5 * gt_w
    gt_cy = by1 + 0.5 * gt_h
    dx_ref[...] = (gt_cx - ex_cx) / ex_w
    dy_ref[...] = (gt_cy - ex_cy) / ex_h
    dw_ref[...] = jnp.log(gt_w / ex_w)
    dh_ref[...] = jnp.log(gt_h / ex_h)


def kernel(im_info, gt_boxes):
    gt = gt_boxes[0].astype(jnp.float32)
    im = im_info.astype(jnp.float32)
    plane = jax.ShapeDtypeStruct((_ROWS, _LANES), jnp.float32)
    lab, dx, dy, dw, dh = pl.pallas_call(
        _tc_body,
        out_shape=[plane] * 5,
        in_specs=[
            pl.BlockSpec(memory_space=pltpu.SMEM),
            pl.BlockSpec(memory_space=pltpu.SMEM),
            pl.BlockSpec(memory_space=pltpu.VMEM),
            pl.BlockSpec(memory_space=pltpu.VMEM),
            pl.BlockSpec(memory_space=pltpu.VMEM),
            pl.BlockSpec(memory_space=pltpu.VMEM),
        ],
    )(gt, im, _AX1, _AY1, _AX2, _AY2)
    labels = lab.reshape(_NPAD)[:_N][None, :]
    targets = jnp.stack([dx.reshape(_NPAD)[:_N], dy.reshape(_NPAD)[:_N],
                         dw.reshape(_NPAD)[:_N], dh.reshape(_NPAD)[:_N]], axis=1)[None]
    anchors = jnp.asarray(_ANCHORS_NP)[None]
    return labels, targets, anchors
